# baseline (speedup 1.0000x reference)
import functools
import jax
import jax.numpy as jnp
from jax import lax
from jax.experimental import pallas as pl
from jax.experimental.pallas import tpu as pltpu
from jax.experimental.pallas import tpu_sc as plsc

_NUM_CLASSES = 1000000
_DIM = 64
_BATCH = 16384

_info = plsc.get_sparse_core_info()
_NC, _NS = _info.num_cores, _info.num_subcores
_NW = _NC * _NS
_B_PER_W = _BATCH // _NW
_CHUNK = 128
_N_CHUNKS = _B_PER_W // _CHUNK

_mesh = plsc.VectorSubcoreMesh(core_axis_name="c", subcore_axis_name="s")


@functools.partial(
    pl.kernel,
    mesh=_mesh,
    out_type=jax.ShapeDtypeStruct((_BATCH, _DIM), jnp.float32),
    scratch_types=[
        pltpu.VMEM((_B_PER_W,), jnp.int32),
        pltpu.VMEM((_B_PER_W,), jnp.int32),
        pltpu.VMEM((_B_PER_W, 2 * _DIM), jnp.float32),
        pltpu.VMEM((_B_PER_W, _DIM), jnp.float32),
        pltpu.SemaphoreType.DMA,
    ],
    compiler_params=pltpu.CompilerParams(use_tc_tiling_on_sc=False),
)
def _embed(y_hbm, table2_hbm, out_hbm, idx_v, q_v, gbuf, rows_v, sem):
    wid = lax.axis_index("s") * _NC + lax.axis_index("c")
    base = wid * _B_PER_W
    pltpu.sync_copy(y_hbm.at[pl.ds(base, _B_PER_W)], idx_v)

    def q_body(g, _):
        sl = pl.ds(g * 16, 16)
        q_v[sl] = jax.lax.shift_right_logical(idx_v[sl], 1)
        return _

    lax.fori_loop(0, _B_PER_W // 16, q_body, None)

    copies = []
    for j in range(_N_CHUNKS):
        copies.append(
            pltpu.async_copy(
                table2_hbm.at[q_v.at[pl.ds(j * _CHUNK, _CHUNK)]],
                gbuf.at[pl.ds(j * _CHUNK, _CHUNK)],
                sem,
            )
        )
    for c in copies:
        c.wait()

    def sel_body(g, _):
        vec = idx_v[pl.ds(g * 16, 16)]
        for k in range(16):
            j = g * 16 + k
            h = jax.lax.rem(vec[k], 2) * _DIM
            for q in range(4):
                rows_v[j, pl.ds(q * 16, 16)] = gbuf[j, pl.ds(h + q * 16, 16)]
        return _

    lax.fori_loop(0, _B_PER_W // 16, sel_body, None)

    pltpu.sync_copy(rows_v, out_hbm.at[pl.ds(base, _B_PER_W)])


def kernel(y, emb_weight):
    assert y.shape == (_BATCH,) and emb_weight.shape == (_NUM_CLASSES, _DIM)
    table2 = emb_weight.reshape(_NUM_CLASSES // 2, 2 * _DIM)
    return _embed(y.astype(jnp.int32), table2)


# speedup vs baseline: 1.0090x; 1.0090x over previous
import functools
import jax
import jax.numpy as jnp
from jax import lax
from jax.experimental import pallas as pl
from jax.experimental.pallas import tpu as pltpu
from jax.experimental.pallas import tpu_sc as plsc

_NUM_CLASSES = 1000000
_DIM = 64
_BATCH = 16384

_info = plsc.get_sparse_core_info()
_NC, _NS = _info.num_cores, _info.num_subcores
_NW = _NC * _NS
_B_PER_W = _BATCH // _NW
_CHUNK = 128
_N_CHUNKS = _B_PER_W // _CHUNK
_PASS_CHUNKS = 2

_mesh = plsc.VectorSubcoreMesh(core_axis_name="c", subcore_axis_name="s")


@functools.partial(
    pl.kernel,
    mesh=_mesh,
    out_type=jax.ShapeDtypeStruct((_BATCH, _DIM), jnp.float32),
    scratch_types=[
        pltpu.VMEM((_B_PER_W,), jnp.int32),
        pltpu.VMEM((_B_PER_W,), jnp.int32),
        pltpu.VMEM((_PASS_CHUNKS * _CHUNK, 2 * _DIM), jnp.float32),
        pltpu.VMEM((_B_PER_W, _DIM), jnp.float32),
        pltpu.SemaphoreType.DMA,
    ],
)
def _embed(y_hbm, table2_hbm, out_hbm, idx_v, q_v, gbuf, rows_v, sem):
    wid = lax.axis_index("s") * _NC + lax.axis_index("c")
    base = wid * _B_PER_W
    pltpu.sync_copy(y_hbm.at[pl.ds(base, _B_PER_W)], idx_v)

    def q_body(g, _):
        sl = pl.ds(g * 16, 16)
        q_v[sl] = jax.lax.shift_right_logical(idx_v[sl], 1)
        return _

    lax.fori_loop(0, _B_PER_W // 16, q_body, None)

    for p in range(_N_CHUNKS // _PASS_CHUNKS):
        pbase = p * _PASS_CHUNKS * _CHUNK
        copies = []
        for j in range(_PASS_CHUNKS):
            copies.append(
                pltpu.async_copy(
                    table2_hbm.at[q_v.at[pl.ds(pbase + j * _CHUNK, _CHUNK)]],
                    gbuf.at[pl.ds(j * _CHUNK, _CHUNK)],
                    sem,
                )
            )
        for c in copies:
            c.wait()

        def sel_body(g, _):
            vec = idx_v[pl.ds(pbase + g * 16, 16)]
            for k in range(16):
                j = g * 16 + k
                h = jax.lax.rem(vec[k], 2) * _DIM
                for q in range(4):
                    rows_v[pbase + j, pl.ds(q * 16, 16)] = gbuf[
                        j, pl.ds(h + q * 16, 16)
                    ]
            return _

        lax.fori_loop(0, _PASS_CHUNKS * _CHUNK // 16, sel_body, None)

    pltpu.sync_copy(rows_v, out_hbm.at[pl.ds(base, _B_PER_W)])


def kernel(y, emb_weight):
    assert y.shape == (_BATCH,) and emb_weight.shape == (_NUM_CLASSES, _DIM)
    table2 = emb_weight.reshape(_NUM_CLASSES // 2, 2 * _DIM)
    return _embed(y.astype(jnp.int32), table2)
